# baseline (speedup 1.0000x reference)
import functools

import jax
import jax.numpy as jnp
from jax import lax
from jax.experimental import pallas as pl
from jax.experimental.pallas import tpu as pltpu
from jax.experimental.pallas import tpu_sc as plsc

EMBED_D = 128
N_LAYERS = 2
N_TYPES = 3
NODE_COUNT = 100000
S = 10
B = 1024
M = 3
BM = M * B
PIECE = S * BM

NC, NS = 2, 16
NW = NC * NS

N_SELF = M * B
SELF_PER_W = N_SELF // NW
CHUNK = 120
NBUF = 6
AHEAD = 3


def _sc_gather(table_flat, nidx3d, self_idx):
    mesh = plsc.VectorSubcoreMesh(core_axis_name="c", subcore_axis_name="s")
    with_self = self_idx is not None
    n_chunks = nidx3d.shape[1]
    rows_per_w = n_chunks * CHUNK

    out_type = [jax.ShapeDtypeStruct((NW * rows_per_w, EMBED_D), jnp.float32)]
    scratch = [
        pltpu.VMEM((1, n_chunks, CHUNK), jnp.int32),
        pltpu.VMEM((NBUF, CHUNK, EMBED_D), jnp.float32),
        [pltpu.SemaphoreType.DMA] * NBUF,
        [pltpu.SemaphoreType.DMA] * NBUF,
    ]
    if with_self:
        out_type.append(jax.ShapeDtypeStruct((N_SELF, EMBED_D), jnp.float32))
        scratch += [
            pltpu.VMEM((SELF_PER_W,), jnp.int32),
            pltpu.VMEM((SELF_PER_W, EMBED_D), jnp.float32),
            pltpu.SemaphoreType.DMA,
        ]

    @functools.partial(pl.kernel, mesh=mesh, out_type=out_type,
                       scratch_types=scratch)
    def k(table_hbm, nidx_hbm, *rest):
        if with_self:
            (sidx_hbm, nout_hbm, sout_hbm,
             idx_v, rows_v, gsem, wsem, idx_s, rows_s, ssem) = rest
        else:
            (nout_hbm, idx_v, rows_v, gsem, wsem) = rest
        wid = lax.axis_index("s") * NC + lax.axis_index("c")

        if with_self:
            sbase = wid * SELF_PER_W
            pltpu.sync_copy(sidx_hbm.at[pl.ds(sbase, SELF_PER_W)], idx_s)
            self_gather = pltpu.async_copy(table_hbm.at[idx_s], rows_s, ssem)

        pltpu.sync_copy(nidx_hbm.at[pl.ds(wid, 1)], idx_v)

        def gather_start(c):
            return pltpu.async_copy(
                table_hbm.at[idx_v.at[0, c]], rows_v.at[c % NBUF],
                gsem[c % NBUF])

        def wb_start(c):
            base = wid * rows_per_w + c * CHUNK
            return pltpu.async_copy(
                rows_v.at[c % NBUF], nout_hbm.at[pl.ds(base, CHUNK)],
                wsem[c % NBUF])

        started = {}
        for c in range(min(AHEAD, n_chunks)):
            started[c] = gather_start(c)
        wbs = {}
        for i in range(n_chunks):
            started[i].wait()
            wbs[i] = wb_start(i)
            c = i + AHEAD
            if c < n_chunks:
                if c >= NBUF:
                    wbs[c - NBUF].wait()
                started[c] = gather_start(c)
        for i in range(max(n_chunks - NBUF, 0), n_chunks):
            wbs[i].wait()

        if with_self:
            self_gather.wait()
            pltpu.sync_copy(rows_s, sout_hbm.at[pl.ds(sbase, SELF_PER_W)])

    if with_self:
        return k(table_flat, nidx3d, self_idx)
    return k(table_flat, nidx3d)[0]


def _rnn_agg(neigh_ref, wih_ref, whh_ref, bsum_ref):
    x = neigh_ref[0].astype(jnp.bfloat16)
    w_ih = wih_ref[0].astype(jnp.bfloat16)
    w_hh = whh_ref[0].astype(jnp.bfloat16)
    b = bsum_ref[0, 0, :]
    dn = (((1,), (1,)), ((), ()))
    xw = lax.dot_general(x, w_ih, dn, preferred_element_type=jnp.float32) + b
    h = jnp.zeros((BM, EMBED_D), jnp.float32)
    acc = jnp.zeros((BM, EMBED_D), jnp.float32)
    for s in range(S):
        g = xw[s * BM:(s + 1) * BM] + lax.dot_general(
            h.astype(jnp.bfloat16), w_hh, dn,
            preferred_element_type=jnp.float32)
        h = jnp.tanh(g)
        acc = acc + h
    return acc * (1.0 / S)


def _attention(cur, a0, a1, a2, sem_ref):
    def attend(rows, w):
        c = cur[rows]
        e1 = a0[rows]
        e2 = a1[rows]
        e3 = a2[rows]
        w1 = w[:EMBED_D].reshape(EMBED_D, 1)
        w2 = w[EMBED_D:].reshape(EMBED_D, 1)
        base = jnp.dot(c, w1, preferred_element_type=jnp.float32)
        l0 = base + jnp.dot(c, w2, preferred_element_type=jnp.float32)
        l1 = base + jnp.dot(e1, w2, preferred_element_type=jnp.float32)
        l2 = base + jnp.dot(e2, w2, preferred_element_type=jnp.float32)
        l3 = base + jnp.dot(e3, w2, preferred_element_type=jnp.float32)
        mx = jnp.maximum(jnp.maximum(l0, l1), jnp.maximum(l2, l3))
        p0 = jnp.exp(l0 - mx)
        p1 = jnp.exp(l1 - mx)
        p2 = jnp.exp(l2 - mx)
        p3 = jnp.exp(l3 - mx)
        tot = p0 + p1 + p2 + p3
        v = (p0 * c + p1 * e1 + p2 * e2 + p3 * e3) / tot
        return jnp.where(v >= 0, v, 0.01 * v)

    w_t0 = sem_ref[0, 0, :]
    w_t1 = sem_ref[1, 0, :]
    return jnp.concatenate(
        [attend(slice(0, B), w_t0), attend(slice(B, BM), w_t1)], axis=0)


_NEIGH_SPEC = pl.BlockSpec((1, PIECE, EMBED_D), lambda t: (t, 0, 0))
_W_SPEC = pl.BlockSpec((1, EMBED_D, EMBED_D), lambda t: (t, 0, 0))
_B_SPEC = pl.BlockSpec((1, 8, EMBED_D), lambda t: (t, 0, 0))
_FULL = pl.BlockSpec((BM, EMBED_D), lambda t: (0, 0))
_SEM_SPEC = pl.BlockSpec((2, 8, 2 * EMBED_D), lambda t: (0, 0, 0))
_OUT = jax.ShapeDtypeStruct((BM, EMBED_D), jnp.float32)


def _params(n):
    return dict(
        grid=(n,),
        compiler_params=pltpu.CompilerParams(
            dimension_semantics=("arbitrary",)))


def _body_agg1(neigh_ref, wih_ref, whh_ref, bsum_ref, out_ref):
    out_ref[...] = _rnn_agg(neigh_ref, wih_ref, whh_ref, bsum_ref)


def _tc_agg1(neigh, wih, whh, bsum):
    return pl.pallas_call(
        _body_agg1,
        in_specs=[_NEIGH_SPEC, _W_SPEC, _W_SPEC, _B_SPEC],
        out_specs=_FULL, out_shape=_OUT, **_params(1),
    )(neigh, wih, whh, bsum)


def _body_agg2(neigh_ref, wih_ref, whh_ref, bsum_ref, out0_ref, out1_ref):
    t = pl.program_id(0)
    agg = _rnn_agg(neigh_ref, wih_ref, whh_ref, bsum_ref)

    @pl.when(t == 0)
    def _():
        out0_ref[...] = agg

    @pl.when(t == 1)
    def _():
        out1_ref[...] = agg


def _tc_agg2(neigh, wih, whh, bsum):
    return pl.pallas_call(
        _body_agg2,
        in_specs=[_NEIGH_SPEC, _W_SPEC, _W_SPEC, _B_SPEC],
        out_specs=[_FULL, _FULL], out_shape=[_OUT, _OUT], **_params(2),
    )(neigh, wih, whh, bsum)


def _body_att2(neigh_ref, a0_ref, cur_ref, sem_ref, wih_ref, whh_ref,
               bsum_ref, out_ref, agg_scr):
    t = pl.program_id(0)
    agg = _rnn_agg(neigh_ref, wih_ref, whh_ref, bsum_ref)

    @pl.when(t == 0)
    def _():
        agg_scr[...] = agg

    @pl.when(t == 1)
    def _():
        out_ref[...] = _attention(cur_ref[...], a0_ref[...], agg_scr[...],
                                  agg, sem_ref)


def _tc_att2(neigh, a0, cur, sem, wih, whh, bsum):
    return pl.pallas_call(
        _body_att2,
        in_specs=[_NEIGH_SPEC, _FULL, _FULL, _SEM_SPEC,
                  _W_SPEC, _W_SPEC, _B_SPEC],
        out_specs=_FULL, out_shape=_OUT,
        scratch_shapes=[pltpu.VMEM((BM, EMBED_D), jnp.float32)],
        **_params(2),
    )(neigh, a0, cur, sem, wih, whh, bsum)


def _body_att1(neigh_ref, a0_ref, a1_ref, cur_ref, sem_ref, wih_ref, whh_ref,
               bsum_ref, out_ref):
    agg = _rnn_agg(neigh_ref, wih_ref, whh_ref, bsum_ref)
    out_ref[...] = _attention(cur_ref[...], a0_ref[...], a1_ref[...],
                              agg, sem_ref)


def _tc_att1(neigh, a0, a1, cur, sem, wih, whh, bsum):
    return pl.pallas_call(
        _body_att1,
        in_specs=[_NEIGH_SPEC, _FULL, _FULL, _FULL, _SEM_SPEC,
                  _W_SPEC, _W_SPEC, _B_SPEC],
        out_specs=_FULL, out_shape=_OUT, **_params(1),
    )(neigh, a0, a1, cur, sem, wih, whh, bsum)


def kernel(c_idx, pos_idx, neg_idx, neigh_c, neigh_pos, neigh_neg,
           tables, W_ih, W_hh, b_ih, b_hh, sem_w):
    table_flat = tables.reshape(N_TYPES * NODE_COUNT, EMBED_D)

    neigh_all = jnp.stack([neigh_c, neigh_pos, neigh_neg], axis=0)
    offs = (jnp.arange(N_TYPES, dtype=neigh_all.dtype) * NODE_COUNT
            ).reshape(1, 1, N_TYPES, 1, 1)
    nidx = jnp.transpose(neigh_all + offs, (1, 2, 4, 0, 3))
    nidx = nidx.reshape(N_LAYERS * N_TYPES * PIECE).astype(jnp.int32)
    sidx = jnp.concatenate(
        [c_idx, pos_idx + NODE_COUNT, neg_idx + NODE_COUNT]).astype(jnp.int32)

    def piece_idx(lo_lt, n_lt):
        p = nidx[lo_lt * PIECE:(lo_lt + n_lt) * PIECE]
        return p.reshape(NW, (n_lt * PIECE) // (NW * CHUNK), CHUNK)

    bsum = jnp.broadcast_to((b_ih + b_hh)[:, :, None, :],
                            (N_LAYERS, N_TYPES, 8, EMBED_D))
    semb = jnp.broadcast_to(sem_w[:, :2, None, :],
                            (N_LAYERS, 2, 8, 2 * EMBED_D))

    g0, cur0 = _sc_gather(table_flat, piece_idx(0, 1), sidx)
    g1 = _sc_gather(table_flat, piece_idx(1, 2), None)
    g2 = _sc_gather(table_flat, piece_idx(3, 2), None)
    g3 = _sc_gather(table_flat, piece_idx(5, 1), None)

    a00 = _tc_agg1(g0.reshape(1, PIECE, EMBED_D),
                   W_ih[0, 0:1], W_hh[0, 0:1], bsum[0, 0:1])
    cur1 = _tc_att2(g1.reshape(2, PIECE, EMBED_D), a00, cur0, semb[0],
                    W_ih[0, 1:3], W_hh[0, 1:3], bsum[0, 1:3])
    a10, a11 = _tc_agg2(g2.reshape(2, PIECE, EMBED_D),
                        W_ih[1, 0:2], W_hh[1, 0:2], bsum[1, 0:2])
    out = _tc_att1(g3.reshape(1, PIECE, EMBED_D), a10, a11, cur1, semb[1],
                   W_ih[1, 2:3], W_hh[1, 2:3], bsum[1, 2:3])

    return (out[:B], out[B:2 * B], out[2 * B:])


# speedup vs baseline: 1.0962x; 1.0962x over previous
import functools

import jax
import jax.numpy as jnp
from jax import lax
from jax.experimental import pallas as pl
from jax.experimental.pallas import tpu as pltpu
from jax.experimental.pallas import tpu_sc as plsc

EMBED_D = 128
N_LAYERS = 2
N_TYPES = 3
NODE_COUNT = 100000
S = 10
B = 1024
M = 3
BM = M * B

NC, NS = 2, 16
NW = NC * NS

N_LAYER_ROWS = N_TYPES * S * M * B
N_SELF = M * B
ROWS_PER_W = N_LAYER_ROWS // NW
SELF_PER_W = N_SELF // NW
CHUNK = 120
N_CHUNKS = ROWS_PER_W // CHUNK
NBUF = 6
AHEAD = 3


def _sc_gather(table_flat, nidx3d, self_idx):
    mesh = plsc.VectorSubcoreMesh(core_axis_name="c", subcore_axis_name="s")
    with_self = self_idx is not None

    out_type = [jax.ShapeDtypeStruct((NW * ROWS_PER_W, EMBED_D), jnp.float32)]
    scratch = [
        pltpu.VMEM((1, N_CHUNKS, CHUNK), jnp.int32),
        pltpu.VMEM((NBUF, CHUNK, EMBED_D), jnp.float32),
        [pltpu.SemaphoreType.DMA] * NBUF,
        [pltpu.SemaphoreType.DMA] * NBUF,
    ]
    if with_self:
        out_type.append(jax.ShapeDtypeStruct((N_SELF, EMBED_D), jnp.float32))
        scratch += [
            pltpu.VMEM((SELF_PER_W,), jnp.int32),
            pltpu.VMEM((SELF_PER_W, EMBED_D), jnp.float32),
            pltpu.SemaphoreType.DMA,
        ]

    @functools.partial(pl.kernel, mesh=mesh, out_type=out_type,
                       scratch_types=scratch)
    def k(table_hbm, nidx_hbm, *rest):
        if with_self:
            (sidx_hbm, nout_hbm, sout_hbm,
             idx_v, rows_v, gsem, wsem, idx_s, rows_s, ssem) = rest
        else:
            (nout_hbm, idx_v, rows_v, gsem, wsem) = rest
        wid = lax.axis_index("s") * NC + lax.axis_index("c")

        if with_self:
            sbase = wid * SELF_PER_W
            pltpu.sync_copy(sidx_hbm.at[pl.ds(sbase, SELF_PER_W)], idx_s)
            self_gather = pltpu.async_copy(table_hbm.at[idx_s], rows_s, ssem)

        pltpu.sync_copy(nidx_hbm.at[pl.ds(wid, 1)], idx_v)

        def gather_start(c):
            return pltpu.async_copy(
                table_hbm.at[idx_v.at[0, c]], rows_v.at[c % NBUF],
                gsem[c % NBUF])

        def wb_start(c):
            base = wid * ROWS_PER_W + c * CHUNK
            return pltpu.async_copy(
                rows_v.at[c % NBUF], nout_hbm.at[pl.ds(base, CHUNK)],
                wsem[c % NBUF])

        started = {}
        for c in range(AHEAD):
            started[c] = gather_start(c)
        wbs = {}
        for i in range(N_CHUNKS):
            started[i].wait()
            wbs[i] = wb_start(i)
            c = i + AHEAD
            if c < N_CHUNKS:
                if c >= NBUF:
                    wbs[c - NBUF].wait()
                started[c] = gather_start(c)
        for i in range(max(N_CHUNKS - NBUF, 0), N_CHUNKS):
            wbs[i].wait()

        if with_self:
            self_gather.wait()
            pltpu.sync_copy(rows_s, sout_hbm.at[pl.ds(sbase, SELF_PER_W)])

    if with_self:
        return k(table_flat, nidx3d, self_idx)
    return k(table_flat, nidx3d)[0]


def _tc_layer_body(neigh_ref, cur_ref, wih_ref, whh_ref, bsum_ref, sem_ref,
                   out_ref, aggs_ref):
    t = pl.program_id(0)

    x = neigh_ref[0].astype(jnp.bfloat16)
    w_ih = wih_ref[0].astype(jnp.bfloat16)
    w_hh = whh_ref[0].astype(jnp.bfloat16)
    b = bsum_ref[0, 0, :]
    dn = (((1,), (1,)), ((), ()))
    xw = lax.dot_general(x, w_ih, dn, preferred_element_type=jnp.float32) + b
    h = jnp.zeros((BM, EMBED_D), jnp.float32)
    acc = jnp.zeros((BM, EMBED_D), jnp.float32)
    for s in range(S):
        g = xw[s * BM:(s + 1) * BM] + lax.dot_general(
            h.astype(jnp.bfloat16), w_hh, dn,
            preferred_element_type=jnp.float32)
        h = jnp.tanh(g)
        acc = acc + h
    agg = acc * (1.0 / S)

    for k in range(N_TYPES):
        @pl.when(t == k)
        def _():
            aggs_ref[k] = agg

    @pl.when(t == N_TYPES - 1)
    def _():
        cur = cur_ref[...]
        a0 = aggs_ref[0]
        a1 = aggs_ref[1]
        a2 = aggs_ref[2]

        def attend(rows, w):
            c = cur[rows]
            e1 = a0[rows]
            e2 = a1[rows]
            e3 = a2[rows]
            w1 = w[:EMBED_D].reshape(EMBED_D, 1)
            w2 = w[EMBED_D:].reshape(EMBED_D, 1)
            base = jnp.dot(c, w1, preferred_element_type=jnp.float32)
            l0 = base + jnp.dot(c, w2, preferred_element_type=jnp.float32)
            l1 = base + jnp.dot(e1, w2, preferred_element_type=jnp.float32)
            l2 = base + jnp.dot(e2, w2, preferred_element_type=jnp.float32)
            l3 = base + jnp.dot(e3, w2, preferred_element_type=jnp.float32)
            mx = jnp.maximum(jnp.maximum(l0, l1), jnp.maximum(l2, l3))
            p0 = jnp.exp(l0 - mx)
            p1 = jnp.exp(l1 - mx)
            p2 = jnp.exp(l2 - mx)
            p3 = jnp.exp(l3 - mx)
            tot = p0 + p1 + p2 + p3
            v = (p0 * c + p1 * e1 + p2 * e2 + p3 * e3) / tot
            return jnp.where(v >= 0, v, 0.01 * v)

        w_t0 = sem_ref[0, 0, :]
        w_t1 = sem_ref[1, 0, :]
        out_ref[...] = jnp.concatenate(
            [attend(slice(0, B), w_t0), attend(slice(B, BM), w_t1)], axis=0)


def _tc_layer(neigh_emb, cur, W_ih_l, W_hh_l, bsum_l, sem_l):
    return pl.pallas_call(
        _tc_layer_body,
        grid=(N_TYPES,),
        in_specs=[
            pl.BlockSpec((1, S * BM, EMBED_D), lambda t: (t, 0, 0)),
            pl.BlockSpec((BM, EMBED_D), lambda t: (0, 0)),
            pl.BlockSpec((1, EMBED_D, EMBED_D), lambda t: (t, 0, 0)),
            pl.BlockSpec((1, EMBED_D, EMBED_D), lambda t: (t, 0, 0)),
            pl.BlockSpec((1, 8, EMBED_D), lambda t: (t, 0, 0)),
            pl.BlockSpec((2, 8, 2 * EMBED_D), lambda t: (0, 0, 0)),
        ],
        out_specs=pl.BlockSpec((BM, EMBED_D), lambda t: (0, 0)),
        out_shape=jax.ShapeDtypeStruct((BM, EMBED_D), jnp.float32),
        scratch_shapes=[pltpu.VMEM((N_TYPES, BM, EMBED_D), jnp.float32)],
        compiler_params=pltpu.CompilerParams(
            dimension_semantics=("arbitrary",)),
    )(neigh_emb, cur, W_ih_l, W_hh_l, bsum_l, sem_l)


def kernel(c_idx, pos_idx, neg_idx, neigh_c, neigh_pos, neigh_neg,
           tables, W_ih, W_hh, b_ih, b_hh, sem_w):
    table_flat = tables.reshape(N_TYPES * NODE_COUNT, EMBED_D)

    neigh_all = jnp.stack([neigh_c, neigh_pos, neigh_neg], axis=0)
    offs = (jnp.arange(N_TYPES, dtype=neigh_all.dtype) * NODE_COUNT
            ).reshape(1, 1, N_TYPES, 1, 1)
    nidx = jnp.transpose(neigh_all + offs, (1, 2, 4, 0, 3))
    nidx = nidx.reshape(N_LAYERS, NW, N_CHUNKS, CHUNK).astype(jnp.int32)
    sidx = jnp.concatenate(
        [c_idx, pos_idx + NODE_COUNT, neg_idx + NODE_COUNT]).astype(jnp.int32)

    bsum = jnp.broadcast_to((b_ih + b_hh)[:, :, None, :],
                            (N_LAYERS, N_TYPES, 8, EMBED_D))
    semb = jnp.broadcast_to(sem_w[:, :2, None, :],
                            (N_LAYERS, 2, 8, 2 * EMBED_D))

    neigh0, cur = _sc_gather(table_flat, nidx[0], sidx)
    neigh1 = _sc_gather(table_flat, nidx[1], None)
    neigh0 = neigh0.reshape(N_TYPES, S * BM, EMBED_D)
    neigh1 = neigh1.reshape(N_TYPES, S * BM, EMBED_D)

    cur = _tc_layer(neigh0, cur, W_ih[0], W_hh[0], bsum[0], semb[0])
    out = _tc_layer(neigh1, cur, W_ih[1], W_hh[1], bsum[1], semb[1])
    return (out[:B], out[B:2 * B], out[2 * B:])


# speedup vs baseline: 1.1063x; 1.0093x over previous
import functools

import jax
import jax.numpy as jnp
from jax import lax
from jax.experimental import pallas as pl
from jax.experimental.pallas import tpu as pltpu
from jax.experimental.pallas import tpu_sc as plsc

EMBED_D = 128
N_LAYERS = 2
N_TYPES = 3
NODE_COUNT = 100000
S = 10
B = 1024
M = 3
BM = M * B

NC, NS = 2, 16
NW = NC * NS

N_LAYER_ROWS = N_TYPES * S * M * B
N_SELF = M * B
ROWS_PER_W = N_LAYER_ROWS // NW
SELF_PER_W = N_SELF // NW
CHUNK = 120
N_CHUNKS = ROWS_PER_W // CHUNK
NBUF = 7
AHEAD = 4


def _sc_gather(table_flat, nidx3d, self_idx):
    mesh = plsc.VectorSubcoreMesh(core_axis_name="c", subcore_axis_name="s")
    with_self = self_idx is not None

    out_type = [jax.ShapeDtypeStruct((NW * ROWS_PER_W, EMBED_D), jnp.float32)]
    scratch = [
        pltpu.VMEM((1, N_CHUNKS, CHUNK), jnp.int32),
        pltpu.VMEM((NBUF, CHUNK, EMBED_D), jnp.float32),
        [pltpu.SemaphoreType.DMA] * NBUF,
        [pltpu.SemaphoreType.DMA] * NBUF,
    ]
    if with_self:
        out_type.append(jax.ShapeDtypeStruct((N_SELF, EMBED_D), jnp.float32))
        scratch += [
            pltpu.VMEM((SELF_PER_W,), jnp.int32),
            pltpu.VMEM((SELF_PER_W, EMBED_D), jnp.float32),
            pltpu.SemaphoreType.DMA,
        ]

    @functools.partial(pl.kernel, mesh=mesh, out_type=out_type,
                       scratch_types=scratch)
    def k(table_hbm, nidx_hbm, *rest):
        if with_self:
            (sidx_hbm, nout_hbm, sout_hbm,
             idx_v, rows_v, gsem, wsem, idx_s, rows_s, ssem) = rest
        else:
            (nout_hbm, idx_v, rows_v, gsem, wsem) = rest
        wid = lax.axis_index("s") * NC + lax.axis_index("c")

        if with_self:
            sbase = wid * SELF_PER_W
            pltpu.sync_copy(sidx_hbm.at[pl.ds(sbase, SELF_PER_W)], idx_s)
            self_gather = pltpu.async_copy(table_hbm.at[idx_s], rows_s, ssem)

        pltpu.sync_copy(nidx_hbm.at[pl.ds(wid, 1)], idx_v)

        def gather_start(c):
            return pltpu.async_copy(
                table_hbm.at[idx_v.at[0, c]], rows_v.at[c % NBUF],
                gsem[c % NBUF])

        def wb_start(c):
            base = wid * ROWS_PER_W + c * CHUNK
            return pltpu.async_copy(
                rows_v.at[c % NBUF], nout_hbm.at[pl.ds(base, CHUNK)],
                wsem[c % NBUF])

        started = {}
        for c in range(AHEAD):
            started[c] = gather_start(c)
        wbs = {}
        for i in range(N_CHUNKS):
            started[i].wait()
            wbs[i] = wb_start(i)
            c = i + AHEAD
            if c < N_CHUNKS:
                if c >= NBUF:
                    wbs[c - NBUF].wait()
                started[c] = gather_start(c)
        for i in range(max(N_CHUNKS - NBUF, 0), N_CHUNKS):
            wbs[i].wait()

        if with_self:
            self_gather.wait()
            pltpu.sync_copy(rows_s, sout_hbm.at[pl.ds(sbase, SELF_PER_W)])

    if with_self:
        return k(table_flat, nidx3d, self_idx)
    return k(table_flat, nidx3d)[0]


def _tc_layer_body(neigh_ref, cur_ref, wih_ref, whh_ref, bsum_ref, sem_ref,
                   out_ref, aggs_ref):
    t = pl.program_id(0)

    x = neigh_ref[0].astype(jnp.bfloat16)
    w_ih = wih_ref[0].astype(jnp.bfloat16)
    w_hh = whh_ref[0].astype(jnp.bfloat16)
    b = bsum_ref[0, 0, :]
    dn = (((1,), (1,)), ((), ()))
    xw = lax.dot_general(x, w_ih, dn, preferred_element_type=jnp.float32) + b
    h = jnp.zeros((BM, EMBED_D), jnp.float32)
    acc = jnp.zeros((BM, EMBED_D), jnp.float32)
    for s in range(S):
        g = xw[s * BM:(s + 1) * BM] + lax.dot_general(
            h.astype(jnp.bfloat16), w_hh, dn,
            preferred_element_type=jnp.float32)
        h = jnp.tanh(g)
        acc = acc + h
    agg = acc * (1.0 / S)

    for k in range(N_TYPES):
        @pl.when(t == k)
        def _():
            aggs_ref[k] = agg

    @pl.when(t == N_TYPES - 1)
    def _():
        cur = cur_ref[...]
        a0 = aggs_ref[0]
        a1 = aggs_ref[1]
        a2 = aggs_ref[2]

        def attend(rows, w):
            c = cur[rows]
            e1 = a0[rows]
            e2 = a1[rows]
            e3 = a2[rows]
            w1 = w[:EMBED_D].reshape(EMBED_D, 1)
            w2 = w[EMBED_D:].reshape(EMBED_D, 1)
            base = jnp.dot(c, w1, preferred_element_type=jnp.float32)
            l0 = base + jnp.dot(c, w2, preferred_element_type=jnp.float32)
            l1 = base + jnp.dot(e1, w2, preferred_element_type=jnp.float32)
            l2 = base + jnp.dot(e2, w2, preferred_element_type=jnp.float32)
            l3 = base + jnp.dot(e3, w2, preferred_element_type=jnp.float32)
            mx = jnp.maximum(jnp.maximum(l0, l1), jnp.maximum(l2, l3))
            p0 = jnp.exp(l0 - mx)
            p1 = jnp.exp(l1 - mx)
            p2 = jnp.exp(l2 - mx)
            p3 = jnp.exp(l3 - mx)
            tot = p0 + p1 + p2 + p3
            v = (p0 * c + p1 * e1 + p2 * e2 + p3 * e3) / tot
            return jnp.where(v >= 0, v, 0.01 * v)

        w_t0 = sem_ref[0, 0, :]
        w_t1 = sem_ref[1, 0, :]
        out_ref[...] = jnp.concatenate(
            [attend(slice(0, B), w_t0), attend(slice(B, BM), w_t1)], axis=0)


def _tc_layer(neigh_emb, cur, W_ih_l, W_hh_l, bsum_l, sem_l):
    return pl.pallas_call(
        _tc_layer_body,
        grid=(N_TYPES,),
        in_specs=[
            pl.BlockSpec((1, S * BM, EMBED_D), lambda t: (t, 0, 0)),
            pl.BlockSpec((BM, EMBED_D), lambda t: (0, 0)),
            pl.BlockSpec((1, EMBED_D, EMBED_D), lambda t: (t, 0, 0)),
            pl.BlockSpec((1, EMBED_D, EMBED_D), lambda t: (t, 0, 0)),
            pl.BlockSpec((1, 8, EMBED_D), lambda t: (t, 0, 0)),
            pl.BlockSpec((2, 8, 2 * EMBED_D), lambda t: (0, 0, 0)),
        ],
        out_specs=pl.BlockSpec((BM, EMBED_D), lambda t: (0, 0)),
        out_shape=jax.ShapeDtypeStruct((BM, EMBED_D), jnp.float32),
        scratch_shapes=[pltpu.VMEM((N_TYPES, BM, EMBED_D), jnp.float32)],
        compiler_params=pltpu.CompilerParams(
            dimension_semantics=("arbitrary",)),
    )(neigh_emb, cur, W_ih_l, W_hh_l, bsum_l, sem_l)


def kernel(c_idx, pos_idx, neg_idx, neigh_c, neigh_pos, neigh_neg,
           tables, W_ih, W_hh, b_ih, b_hh, sem_w):
    table_flat = tables.reshape(N_TYPES * NODE_COUNT, EMBED_D)

    neigh_all = jnp.stack([neigh_c, neigh_pos, neigh_neg], axis=0)
    offs = (jnp.arange(N_TYPES, dtype=neigh_all.dtype) * NODE_COUNT
            ).reshape(1, 1, N_TYPES, 1, 1)
    nidx = jnp.transpose(neigh_all + offs, (1, 2, 4, 0, 3))
    nidx = nidx.reshape(N_LAYERS, NW, N_CHUNKS, CHUNK).astype(jnp.int32)
    sidx = jnp.concatenate(
        [c_idx, pos_idx + NODE_COUNT, neg_idx + NODE_COUNT]).astype(jnp.int32)

    bsum = jnp.broadcast_to((b_ih + b_hh)[:, :, None, :],
                            (N_LAYERS, N_TYPES, 8, EMBED_D))
    semb = jnp.broadcast_to(sem_w[:, :2, None, :],
                            (N_LAYERS, 2, 8, 2 * EMBED_D))

    neigh0, cur = _sc_gather(table_flat, nidx[0], sidx)
    neigh1 = _sc_gather(table_flat, nidx[1], None)
    neigh0 = neigh0.reshape(N_TYPES, S * BM, EMBED_D)
    neigh1 = neigh1.reshape(N_TYPES, S * BM, EMBED_D)

    cur = _tc_layer(neigh0, cur, W_ih[0], W_hh[0], bsum[0], semb[0])
    out = _tc_layer(neigh1, cur, W_ih[1], W_hh[1], bsum[1], semb[1])
    return (out[:B], out[B:2 * B], out[2 * B:])
